# baseline (speedup 1.0000x reference)
import jax
import jax.numpy as jnp
from jax.experimental import pallas as pl
from jax.experimental.pallas import tpu as pltpu

D_MODEL = 768
NUM_EXPERTS = 64
N_TOKENS = 128


def _moe_kernel(x_ref, gate_ref, w1_ref, b1_ref, w2_ref, b2_ref, out_ref,
                h_ref, i1_ref, i2_ref, p1_ref, p2_ref):
    e = pl.program_id(0)
    p = pl.program_id(1)

    @pl.when((e == 0) & (p == 0))
    def _gating():
        x = x_ref[...]
        logits = jax.lax.dot_general(
            x, gate_ref[...], (((1,), (1,)), ((), ())),
            preferred_element_type=jnp.float32)
        eids = jax.lax.broadcasted_iota(jnp.int32, (N_TOKENS, NUM_EXPERTS), 1)
        big = jnp.int32(NUM_EXPERTS + 1)
        v1 = jnp.max(logits, axis=1, keepdims=True)
        i1 = jnp.min(jnp.where(logits == v1, eids, big), axis=1, keepdims=True)
        masked = jnp.where(eids == i1, -jnp.inf, logits)
        v2 = jnp.max(masked, axis=1, keepdims=True)
        i2 = jnp.min(jnp.where(masked == v2, eids, big), axis=1, keepdims=True)
        t = jnp.exp(v2 - v1)
        i1_ref[...] = i1
        i2_ref[...] = i2
        p1_ref[...] = 1.0 / (1.0 + t)
        p2_ref[...] = t / (1.0 + t)

    @pl.when(p == 0)
    def _ffn_in():
        xb = x_ref[...].astype(jnp.bfloat16)
        w1 = w1_ref[0].astype(jnp.bfloat16)
        h = jax.lax.dot_general(xb, w1, (((1,), (0,)), ((), ())),
                                preferred_element_type=jnp.float32)
        h += b1_ref[0, 0]
        h = h * 0.5 * (1.0 + jax.lax.erf(h * 0.7071067811865476))
        h_ref[...] = h.astype(jnp.bfloat16)

    @pl.when(p == 1)
    def _ffn_out():
        w2 = w2_ref[0].astype(jnp.bfloat16)
        o = jax.lax.dot_general(h_ref[...], w2, (((1,), (0,)), ((), ())),
                                preferred_element_type=jnp.float32)
        o += b2_ref[0, 0]
        wcol = (jnp.where(i1_ref[...] == e, p1_ref[...], 0.0)
                + jnp.where(i2_ref[...] == e, p2_ref[...], 0.0))
        contrib = o * wcol

        @pl.when(e == 0)
        def _init():
            out_ref[...] = contrib

        @pl.when(e != 0)
        def _acc():
            out_ref[...] += contrib


@jax.jit
def kernel(x, gate_w, w1, b1, w2, b2):
    Bs, Ts, D = x.shape
    x_flat = x.reshape(-1, D)
    out = pl.pallas_call(
        _moe_kernel,
        grid=(NUM_EXPERTS, 2),
        in_specs=[
            pl.BlockSpec((N_TOKENS, D_MODEL), lambda e, p: (0, 0)),
            pl.BlockSpec((NUM_EXPERTS, D_MODEL), lambda e, p: (0, 0)),
            pl.BlockSpec((1, D_MODEL, D_MODEL), lambda e, p: (e, 0, 0)),
            pl.BlockSpec((1, 1, D_MODEL), lambda e, p: (e, 0, 0)),
            pl.BlockSpec((1, D_MODEL, D_MODEL), lambda e, p: (e, 0, 0)),
            pl.BlockSpec((1, 1, D_MODEL), lambda e, p: (e, 0, 0)),
        ],
        out_specs=pl.BlockSpec((N_TOKENS, D_MODEL), lambda e, p: (0, 0)),
        out_shape=jax.ShapeDtypeStruct((N_TOKENS, D_MODEL), jnp.float32),
        scratch_shapes=[pltpu.VMEM((N_TOKENS, D_MODEL), jnp.bfloat16),
                        pltpu.VMEM((N_TOKENS, 1), jnp.int32),
                        pltpu.VMEM((N_TOKENS, 1), jnp.int32),
                        pltpu.VMEM((N_TOKENS, 1), jnp.float32),
                        pltpu.VMEM((N_TOKENS, 1), jnp.float32)],
    )(x_flat, gate_w, w1, b1[:, None, :], w2, b2[:, None, :])
    return out.reshape(Bs, Ts, D)


# speedup vs baseline: 1.6567x; 1.6567x over previous
import jax
import jax.numpy as jnp
from jax.experimental import pallas as pl
from jax.experimental.pallas import tpu as pltpu

D_MODEL = 768
NUM_EXPERTS = 64
N_TOKENS = 128


def _moe_kernel(x_ref, gate_ref, w1_ref, b1_ref, w2_ref, b2_ref, out_ref,
                i1_ref, i2_ref, p1_ref, p2_ref):
    e = pl.program_id(0)

    @pl.when(e == 0)
    def _gating():
        x = x_ref[...]
        logits = jax.lax.dot_general(
            x, gate_ref[...], (((1,), (1,)), ((), ())),
            preferred_element_type=jnp.float32)
        eids = jax.lax.broadcasted_iota(jnp.int32, (N_TOKENS, NUM_EXPERTS), 1)
        big = jnp.int32(NUM_EXPERTS + 1)
        v1 = jnp.max(logits, axis=1, keepdims=True)
        i1 = jnp.min(jnp.where(logits == v1, eids, big), axis=1, keepdims=True)
        masked = jnp.where(eids == i1, -jnp.inf, logits)
        v2 = jnp.max(masked, axis=1, keepdims=True)
        i2 = jnp.min(jnp.where(masked == v2, eids, big), axis=1, keepdims=True)
        t = jnp.exp(v2 - v1)
        i1_ref[...] = i1
        i2_ref[...] = i2
        p1_ref[...] = 1.0 / (1.0 + t)
        p2_ref[...] = t / (1.0 + t)

    xb = x_ref[...].astype(jnp.bfloat16)
    w1 = w1_ref[0].astype(jnp.bfloat16)
    h = jax.lax.dot_general(xb, w1, (((1,), (0,)), ((), ())),
                            preferred_element_type=jnp.float32)
    h += b1_ref[0, 0]
    h = h * 0.5 * (1.0 + jax.lax.erf(h * 0.7071067811865476))
    w2 = w2_ref[0].astype(jnp.bfloat16)
    o = jax.lax.dot_general(h.astype(jnp.bfloat16), w2,
                            (((1,), (0,)), ((), ())),
                            preferred_element_type=jnp.float32)
    o += b2_ref[0, 0]
    wcol = (jnp.where(i1_ref[...] == e, p1_ref[...], 0.0)
            + jnp.where(i2_ref[...] == e, p2_ref[...], 0.0))
    contrib = o * wcol

    @pl.when(e == 0)
    def _init():
        out_ref[...] = contrib

    @pl.when(e != 0)
    def _acc():
        out_ref[...] += contrib


@jax.jit
def kernel(x, gate_w, w1, b1, w2, b2):
    Bs, Ts, D = x.shape
    x_flat = x.reshape(-1, D)
    out = pl.pallas_call(
        _moe_kernel,
        grid=(NUM_EXPERTS,),
        in_specs=[
            pl.BlockSpec((N_TOKENS, D_MODEL), lambda e: (0, 0)),
            pl.BlockSpec((NUM_EXPERTS, D_MODEL), lambda e: (0, 0)),
            pl.BlockSpec((1, D_MODEL, D_MODEL), lambda e: (e, 0, 0)),
            pl.BlockSpec((1, 1, D_MODEL), lambda e: (e, 0, 0)),
            pl.BlockSpec((1, D_MODEL, D_MODEL), lambda e: (e, 0, 0)),
            pl.BlockSpec((1, 1, D_MODEL), lambda e: (e, 0, 0)),
        ],
        out_specs=pl.BlockSpec((N_TOKENS, D_MODEL), lambda e: (0, 0)),
        out_shape=jax.ShapeDtypeStruct((N_TOKENS, D_MODEL), jnp.float32),
        scratch_shapes=[pltpu.VMEM((N_TOKENS, 1), jnp.int32),
                        pltpu.VMEM((N_TOKENS, 1), jnp.int32),
                        pltpu.VMEM((N_TOKENS, 1), jnp.float32),
                        pltpu.VMEM((N_TOKENS, 1), jnp.float32)],
    )(x_flat, gate_w, w1, b1[:, None, :], w2, b2[:, None, :])
    return out.reshape(Bs, Ts, D)


# speedup vs baseline: 1.8786x; 1.1340x over previous
import jax
import jax.numpy as jnp
from jax.experimental import pallas as pl
from jax.experimental.pallas import tpu as pltpu

D_MODEL = 768
NUM_EXPERTS = 64
N_TOKENS = 128
E_BLK = 2
N_STEPS = NUM_EXPERTS // E_BLK


def _moe_kernel(x_ref, gate_ref, w1_ref, b1_ref, w2_ref, b2_ref, out_ref,
                acc_ref, i1_ref, i2_ref, p1_ref, p2_ref):
    s = pl.program_id(0)

    @pl.when(s == 0)
    def _gating():
        x = x_ref[...]
        logits = jax.lax.dot_general(
            x, gate_ref[...], (((1,), (1,)), ((), ())),
            preferred_element_type=jnp.float32)
        eids = jax.lax.broadcasted_iota(jnp.int32, (N_TOKENS, NUM_EXPERTS), 1)
        big = jnp.int32(NUM_EXPERTS + 1)
        v1 = jnp.max(logits, axis=1, keepdims=True)
        i1 = jnp.min(jnp.where(logits == v1, eids, big), axis=1, keepdims=True)
        masked = jnp.where(eids == i1, -jnp.inf, logits)
        v2 = jnp.max(masked, axis=1, keepdims=True)
        i2 = jnp.min(jnp.where(masked == v2, eids, big), axis=1, keepdims=True)
        t = jnp.exp(v2 - v1)
        i1_ref[...] = i1
        i2_ref[...] = i2
        p1_ref[...] = 1.0 / (1.0 + t)
        p2_ref[...] = t / (1.0 + t)
        acc_ref[...] = jnp.zeros_like(acc_ref)

    xb = x_ref[...].astype(jnp.bfloat16)
    contrib = acc_ref[...]
    for k in range(E_BLK):
        e = s * E_BLK + k
        w1 = w1_ref[k].astype(jnp.bfloat16)
        h = jax.lax.dot_general(xb, w1, (((1,), (0,)), ((), ())),
                                preferred_element_type=jnp.float32)
        h += b1_ref[k, 0]
        h = h * 0.5 * (1.0 + jax.lax.erf(h * 0.7071067811865476))
        w2 = w2_ref[k].astype(jnp.bfloat16)
        o = jax.lax.dot_general(h.astype(jnp.bfloat16), w2,
                                (((1,), (0,)), ((), ())),
                                preferred_element_type=jnp.float32)
        o += b2_ref[k, 0]
        wcol = (jnp.where(i1_ref[...] == e, p1_ref[...], 0.0)
                + jnp.where(i2_ref[...] == e, p2_ref[...], 0.0))
        contrib += o * wcol
    acc_ref[...] = contrib

    @pl.when(s == N_STEPS - 1)
    def _write():
        out_ref[...] = contrib


@jax.jit
def kernel(x, gate_w, w1, b1, w2, b2):
    Bs, Ts, D = x.shape
    x_flat = x.reshape(-1, D)
    out = pl.pallas_call(
        _moe_kernel,
        grid=(N_STEPS,),
        in_specs=[
            pl.BlockSpec((N_TOKENS, D_MODEL), lambda s: (0, 0)),
            pl.BlockSpec((NUM_EXPERTS, D_MODEL), lambda s: (0, 0)),
            pl.BlockSpec((E_BLK, D_MODEL, D_MODEL), lambda s: (s, 0, 0)),
            pl.BlockSpec((E_BLK, 1, D_MODEL), lambda s: (s, 0, 0)),
            pl.BlockSpec((E_BLK, D_MODEL, D_MODEL), lambda s: (s, 0, 0)),
            pl.BlockSpec((E_BLK, 1, D_MODEL), lambda s: (s, 0, 0)),
        ],
        out_specs=pl.BlockSpec((N_TOKENS, D_MODEL), lambda s: (0, 0)),
        out_shape=jax.ShapeDtypeStruct((N_TOKENS, D_MODEL), jnp.float32),
        scratch_shapes=[pltpu.VMEM((N_TOKENS, D_MODEL), jnp.float32),
                        pltpu.VMEM((N_TOKENS, 1), jnp.int32),
                        pltpu.VMEM((N_TOKENS, 1), jnp.int32),
                        pltpu.VMEM((N_TOKENS, 1), jnp.float32),
                        pltpu.VMEM((N_TOKENS, 1), jnp.float32)],
    )(x_flat, gate_w, w1, b1[:, None, :], w2, b2[:, None, :])
    return out.reshape(Bs, Ts, D)
